# baseline (speedup 1.0000x reference)
import functools

import jax
import jax.numpy as jnp
from jax import lax
from jax.experimental import pallas as pl
from jax.experimental.pallas import tpu as pltpu
from jax.experimental.pallas import tpu_sc as plsc

_DIM = 1024
_ROWS = 8192
_SCALE = _DIM ** (-0.5)
_NC = 2
_NS = 16
_NW = _NC * _NS
_ROWS_W = _ROWS // _NW
_NBUF = 6
_CROWS = 8
_NCHUNK = _ROWS_W // _CROWS
_LANES = 16

_mesh = plsc.VectorSubcoreMesh(core_axis_name="c", subcore_axis_name="s")


@functools.partial(
    pl.kernel,
    mesh=_mesh,
    out_type=jax.ShapeDtypeStruct((_ROWS, _DIM), jnp.float32),
    scratch_types=(
        [pltpu.VMEM((_CROWS, _DIM), jnp.float32)] * (2 * _NBUF)
        + [pltpu.SemaphoreType.DMA] * (2 * _NBUF)
    ),
)
def _sc_scale(emb_hbm, out_hbm, *sc):
    bins = sc[:_NBUF]
    bouts = sc[_NBUF : 2 * _NBUF]
    sis = sc[2 * _NBUF : 3 * _NBUF]
    sos = sc[3 * _NBUF : 4 * _NBUF]
    wid = lax.axis_index("s") * _NC + lax.axis_index("c")
    row0 = wid * _ROWS_W

    def _scale_chunk(src, dst):
        group = 8

        def body_r(r, _):
            def body_c(c, _):
                base = c * (group * _LANES)
                vals = [src[r, pl.ds(base + k * _LANES, _LANES)] for k in range(group)]
                vals = [v * _SCALE for v in vals]
                for k in range(group):
                    dst[r, pl.ds(base + k * _LANES, _LANES)] = vals[k]
                return _

            lax.fori_loop(0, _DIM // (group * _LANES), body_c, None)
            return _

        lax.fori_loop(0, _CROWS, body_r, None)

    def _in_slice(g):
        return emb_hbm.at[pl.ds(row0 + g * _CROWS, _CROWS)]

    def _out_slice(g):
        return out_hbm.at[pl.ds(row0 + g * _CROWS, _CROWS)]

    def _start_in(g, b):
        pltpu.async_copy(_in_slice(g), bins[b], sis[b])

    def _wait_in(g, b):
        pltpu.make_async_copy(_in_slice(g), bins[b], sis[b]).wait()

    def _start_out(g, b):
        pltpu.async_copy(bouts[b], _out_slice(g), sos[b])

    def _wait_out(g, b):
        pltpu.make_async_copy(bouts[b], _out_slice(g), sos[b]).wait()

    for b in range(_NBUF):
        _start_in(b, b)
    for b in range(_NBUF):
        _wait_in(b, b)
        _scale_chunk(bins[b], bouts[b])
        _start_out(b, b)
        _start_in(b + _NBUF, b)

    def ring_body(t, _):
        g0 = t * _NBUF
        for b in range(_NBUF):
            g = g0 + b
            _wait_in(g, b)
            _wait_out(g - _NBUF, b)
            _scale_chunk(bins[b], bouts[b])
            _start_out(g, b)
        for b in range(_NBUF):
            g = g0 + b

            @pl.when(g + _NBUF < _NCHUNK)
            def _():
                _start_in(g + _NBUF, b)

        return _

    lax.fori_loop(1, _NCHUNK // _NBUF, ring_body, None)

    for g in range((_NCHUNK // _NBUF) * _NBUF, _NCHUNK):
        b = g % _NBUF
        _wait_in(g, b)
        _wait_out(g - _NBUF, b)
        _scale_chunk(bins[b], bouts[b])
        _start_out(g, b)
    for g in range(_NCHUNK - _NBUF, _NCHUNK):
        _wait_out(g, g % _NBUF)


def kernel(x, emb):
    del x
    return _sc_scale(emb)


# speedup vs baseline: 1.2024x; 1.2024x over previous
import functools

import jax
import jax.numpy as jnp
from jax import lax
from jax.experimental import pallas as pl
from jax.experimental.pallas import tpu as pltpu
from jax.experimental.pallas import tpu_sc as plsc

_DIM = 1024
_ROWS = 8192
_SCALE = _DIM ** (-0.5)
_NC = 2
_NS = 16
_NW = _NC * _NS
_ROWS_W = _ROWS // _NW
_NBUF = 6
_CROWS = 8
_NCHUNK = _ROWS_W // _CROWS
_LANES = 16

_mesh = plsc.VectorSubcoreMesh(core_axis_name="c", subcore_axis_name="s")


@functools.partial(
    pl.kernel,
    mesh=_mesh,
    out_type=jax.ShapeDtypeStruct((_ROWS, _DIM), jnp.float32),
    scratch_types=(
        [pltpu.VMEM((_CROWS, _DIM), jnp.float32)] * (2 * _NBUF)
        + [pltpu.SemaphoreType.DMA] * (2 * _NBUF)
    ),
)
def _sc_scale(emb_hbm, out_hbm, *sc):
    bins = sc[:_NBUF]
    bouts = sc[_NBUF : 2 * _NBUF]
    sis = sc[2 * _NBUF : 3 * _NBUF]
    sos = sc[3 * _NBUF : 4 * _NBUF]
    wid = lax.axis_index("s") * _NC + lax.axis_index("c")
    row0 = wid * _ROWS_W

    def _scale_chunk(src, dst):
        group = 8

        def body_r(r, _):
            def body_c(c, _):
                base = c * (group * _LANES)
                vals = [src[r, pl.ds(base + k * _LANES, _LANES)] for k in range(group)]
                vals = [v * _SCALE for v in vals]
                for k in range(group):
                    dst[r, pl.ds(base + k * _LANES, _LANES)] = vals[k]
                return _

            lax.fori_loop(0, _DIM // (group * _LANES), body_c, None)
            return _

        lax.fori_loop(0, _CROWS, body_r, None)

    def _in_slice(g):
        return emb_hbm.at[pl.ds(row0 + g * _CROWS, _CROWS)]

    def _out_slice(g):
        return out_hbm.at[pl.ds(row0 + g * _CROWS, _CROWS)]

    def _start_in(g, b):
        pltpu.async_copy(_in_slice(g), bins[b], sis[b])

    def _wait_in(g, b):
        pltpu.make_async_copy(_in_slice(g), bins[b], sis[b]).wait()

    def _start_out(g, b):
        pltpu.async_copy(bouts[b], _out_slice(g), sos[b])

    def _wait_out(g, b):
        pltpu.make_async_copy(bouts[b], _out_slice(g), sos[b]).wait()

    for b in range(_NBUF):
        _start_in(b, b)
    for b in range(_NBUF):
        _wait_in(b, b)
        _scale_chunk(bins[b], bouts[b])
        _start_out(b, b)
        _start_in(b + _NBUF, b)

    def ring_body(t, _):
        g0 = t * _NBUF
        for b in range(_NBUF):
            g = g0 + b
            _wait_in(g, b)
            _wait_out(g - _NBUF, b)
            _scale_chunk(bins[b], bouts[b])

            @pl.when(g + _NBUF < _NCHUNK)
            def _():
                _start_in(g + _NBUF, b)

            _start_out(g, b)
        return _

    lax.fori_loop(1, _NCHUNK // _NBUF, ring_body, None)

    for g in range((_NCHUNK // _NBUF) * _NBUF, _NCHUNK):
        b = g % _NBUF
        _wait_in(g, b)
        _wait_out(g - _NBUF, b)
        _scale_chunk(bins[b], bouts[b])
        _start_out(g, b)
    for g in range(_NCHUNK - _NBUF, _NCHUNK):
        _wait_out(g, g % _NBUF)


def kernel(x, emb):
    del x
    return _sc_scale(emb)
